# baseline (speedup 1.0000x reference)
import jax
import jax.numpy as jnp
from jax.experimental import pallas as pl
from jax.experimental.pallas import tpu as pltpu

_B = 16384
_D = 32
_NCB = 8
_K = 1024
_BLK = 2048


def _rvq_body(z_ref, cb_ref, q_ref, idx_ref, cbm2_ref, c2_ref, packed_ref):
    @pl.when(pl.program_id(0) == 0)
    def _init():
        for i in range(_NCB):
            cb = cb_ref[i]
            cbm2_ref[i] = (cb * -2.0).astype(jnp.bfloat16)
            c2_ref[i] = jnp.sum(cb * cb, axis=1)
            p1 = cb.astype(jnp.bfloat16)
            r1 = cb - p1.astype(jnp.float32)
            p2 = r1.astype(jnp.bfloat16)
            p3 = (r1 - p2.astype(jnp.float32)).astype(jnp.bfloat16)
            packed_ref[i] = jnp.concatenate([p1, p2, p3], axis=1)

    _H = _BLK // 2
    iota_k = jax.lax.broadcasted_iota(jnp.int32, (_H, _K), 1)
    for h in range(2):
        rows = pl.ds(h * _H, _H)
        r = z_ref[rows, :]
        q = jnp.zeros_like(r)
        idx_cols = []
        for i in range(_NCB):
            prod2 = jax.lax.dot_general(
                r.astype(jnp.bfloat16), cbm2_ref[i], (((1,), (1,)), ((), ())),
                preferred_element_type=jnp.float32,
            )
            r2 = jnp.sum(r * r, axis=1, keepdims=True)
            d = (r2 + c2_ref[i][None, :]) + prod2
            dmin = jnp.min(d, axis=1, keepdims=True)
            idx = jnp.min(jnp.where(d == dmin, iota_k, _K), axis=1,
                          keepdims=True)
            onehot = (iota_k == idx).astype(jnp.bfloat16)
            g3 = jax.lax.dot_general(
                onehot, packed_ref[i], (((1,), (0,)), ((), ())),
                preferred_element_type=jnp.float32,
            )
            g = (g3[:, :_D] + g3[:, _D:2 * _D]) + g3[:, 2 * _D:]
            r = r - g
            q = q + g
            idx_cols.append(idx)
        q_ref[rows, :] = q
        idx_ref[rows, :] = jnp.concatenate(idx_cols, axis=1)


def kernel(z, codebooks):
    q, idx = pl.pallas_call(
        _rvq_body,
        grid=(_B // _BLK,),
        in_specs=[
            pl.BlockSpec((_BLK, _D), lambda i: (i, 0)),
            pl.BlockSpec((_NCB, _K, _D), lambda i: (0, 0, 0)),
        ],
        out_specs=[
            pl.BlockSpec((_BLK, _D), lambda i: (i, 0)),
            pl.BlockSpec((_BLK, _NCB), lambda i: (i, 0)),
        ],
        out_shape=[
            jax.ShapeDtypeStruct((_B, _D), jnp.float32),
            jax.ShapeDtypeStruct((_B, _NCB), jnp.int32),
        ],
        scratch_shapes=[
            pltpu.VMEM((_NCB, _K, _D), jnp.bfloat16),
            pltpu.VMEM((_NCB, _K), jnp.float32),
            pltpu.VMEM((_NCB, _K, 3 * _D), jnp.bfloat16),
        ],
    )(z, codebooks)
    loss = jnp.zeros((), dtype=jnp.float32)
    return q, loss, idx.astype(jnp.int64)


# speedup vs baseline: 1.0437x; 1.0437x over previous
import jax
import jax.numpy as jnp
from jax.experimental import pallas as pl
from jax.experimental.pallas import tpu as pltpu

_B = 16384
_D = 32
_NCB = 8
_K = 1024
_BLK = 2048


def _rvq_body(z_ref, cb_ref, q_ref, idx_ref, cbm2_ref, c2_ref, packed_ref):
    @pl.when(pl.program_id(0) == 0)
    def _init():
        for i in range(_NCB):
            cb = cb_ref[i]
            cbm2_ref[i] = (cb * -2.0).astype(jnp.bfloat16)
            c2_ref[i] = jnp.sum(cb * cb, axis=1)
            p1 = cb.astype(jnp.bfloat16)
            r1 = cb - p1.astype(jnp.float32)
            p2 = r1.astype(jnp.bfloat16)
            p3 = (r1 - p2.astype(jnp.float32)).astype(jnp.bfloat16)
            packed_ref[i] = jnp.concatenate([p1, p2, p3], axis=1)

    _H = _BLK
    iota_k = jax.lax.broadcasted_iota(jnp.int32, (_H, _K), 1)
    for h in range(1):
        rows = pl.ds(h * _H, _H)
        r = z_ref[rows, :]
        q = jnp.zeros_like(r)
        idx_cols = []
        for i in range(_NCB):
            prod2 = jax.lax.dot_general(
                r.astype(jnp.bfloat16), cbm2_ref[i], (((1,), (1,)), ((), ())),
                preferred_element_type=jnp.float32,
            )
            r2 = jnp.sum(r * r, axis=1, keepdims=True)
            d = (r2 + c2_ref[i][None, :]) + prod2
            dmin = jnp.min(d, axis=1, keepdims=True)
            idx = jnp.min(jnp.where(d == dmin, iota_k, _K), axis=1,
                          keepdims=True)
            onehot = (iota_k == idx).astype(jnp.bfloat16)
            g3 = jax.lax.dot_general(
                onehot, packed_ref[i], (((1,), (0,)), ((), ())),
                preferred_element_type=jnp.float32,
            )
            g = (g3[:, :_D] + g3[:, _D:2 * _D]) + g3[:, 2 * _D:]
            r = r - g
            q = q + g
            idx_cols.append(idx)
        q_ref[rows, :] = q
        idx_ref[rows, :] = jnp.concatenate(idx_cols, axis=1)


def kernel(z, codebooks):
    q, idx = pl.pallas_call(
        _rvq_body,
        grid=(_B // _BLK,),
        in_specs=[
            pl.BlockSpec((_BLK, _D), lambda i: (i, 0)),
            pl.BlockSpec((_NCB, _K, _D), lambda i: (0, 0, 0)),
        ],
        out_specs=[
            pl.BlockSpec((_BLK, _D), lambda i: (i, 0)),
            pl.BlockSpec((_BLK, _NCB), lambda i: (i, 0)),
        ],
        out_shape=[
            jax.ShapeDtypeStruct((_B, _D), jnp.float32),
            jax.ShapeDtypeStruct((_B, _NCB), jnp.int32),
        ],
        scratch_shapes=[
            pltpu.VMEM((_NCB, _K, _D), jnp.bfloat16),
            pltpu.VMEM((_NCB, _K), jnp.float32),
            pltpu.VMEM((_NCB, _K, 3 * _D), jnp.bfloat16),
        ],
    )(z, codebooks)
    loss = jnp.zeros((), dtype=jnp.float32)
    return q, loss, idx.astype(jnp.int64)


# speedup vs baseline: 1.0813x; 1.0360x over previous
import functools

import jax
import jax.numpy as jnp
from jax import lax
from jax.experimental import pallas as pl
from jax.experimental.pallas import tpu as pltpu
from jax.experimental.pallas import tpu_sc as plsc

_B = 16384
_D = 32
_NCB = 8
_K = 1024

_S = 768
_BTC = _B - _S
_BLK = _BTC // 8

_NW = 32
_SP = _S // _NW
_G = 4
_KB = 4
_NKCB = _K // (16 * _KB)


def _rvq_body(z_ref, cb_ref, q_ref, idx_ref):
    _H = _BLK
    iota_k = jax.lax.broadcasted_iota(jnp.int32, (_H, _K), 1)
    cbs, c2s, packeds = [], [], []
    for i in range(_NCB):
        cb = cb_ref[i]
        cbs.append(cb)
        c2s.append(jnp.sum(cb * cb, axis=1))
        p1 = cb.astype(jnp.bfloat16)
        r1 = cb - p1.astype(jnp.float32)
        p2 = r1.astype(jnp.bfloat16)
        p3 = (r1 - p2.astype(jnp.float32)).astype(jnp.bfloat16)
        packeds.append(jnp.concatenate([p1, p2, p3], axis=1))
    for h in range(1):
        rows = pl.ds(h * _H, _H)
        r = z_ref[rows, :]
        q = jnp.zeros_like(r)
        idx_cols = []
        for i in range(_NCB):
            prod = jax.lax.dot_general(
                r.astype(jnp.bfloat16), cbs[i].astype(jnp.bfloat16),
                (((1,), (1,)), ((), ())),
                preferred_element_type=jnp.float32,
            )
            r2 = jnp.sum(r * r, axis=1, keepdims=True)
            d = (r2 + c2s[i][None, :]) - 2.0 * prod
            dmin = jnp.min(d, axis=1, keepdims=True)
            idx = jnp.min(jnp.where(d == dmin, iota_k, _K), axis=1,
                          keepdims=True)
            onehot = (iota_k == idx).astype(jnp.bfloat16)
            g3 = jax.lax.dot_general(
                onehot, packeds[i], (((1,), (0,)), ((), ())),
                preferred_element_type=jnp.float32,
            )
            g = (g3[:, :_D] + g3[:, _D:2 * _D]) + g3[:, 2 * _D:]
            r = r - g
            q = q + g
            idx_cols.append(idx)
        q_ref[rows, :] = q
        idx_ref[rows, :] = jnp.concatenate(idx_cols, axis=1)


def _sc_prep_body(cb_ref, ct_ref, c2_ref, cbf_ref):
    cb = cb_ref[...]
    ct = ((cb * -2.0).astype(jnp.bfloat16).astype(jnp.float32))
    ct = jnp.transpose(ct, (0, 2, 1))
    ct_ref[...] = ct.reshape(_NCB, _D * _K)
    c2_ref[...] = jnp.sum(cb * cb, axis=2)
    cbf_ref[...] = cb.reshape(_NCB, _K * _D)


def _rtne_bf16_value(v):
    u = lax.bitcast_convert_type(v, jnp.int32)
    lsb = jnp.bitwise_and(lax.shift_right_logical(u, 16), 1)
    u2 = u + (lsb + jnp.int32(0x7FFF))
    u3 = jnp.bitwise_and(u2, jnp.int32(-65536))
    return lax.bitcast_convert_type(u3, jnp.float32)


def _sc_body(z_hbm, ct_hbm, c2_hbm, cbf_hbm, q_hbm, idx_hbm,
             rv, rbv, qv, ctv, cbv, c2v, ctv2, c2v2,
             idxv, rotf, roti, sems):
    wid = lax.axis_index("s") * 2 + lax.axis_index("c")
    base = wid * _SP
    lane = lax.iota(jnp.int32, 16)

    bufs = ((ctv, c2v), (ctv2, c2v2))

    def start_fetch(i, buf, sem2):
        pltpu.async_copy(ct_hbm.at[i], buf[0], sems.at[sem2])
        pltpu.async_copy(c2_hbm.at[i], buf[1], sems.at[sem2 + 1])

    def wait_fetch(i, buf, sem2):
        pltpu.make_async_copy(ct_hbm.at[i], buf[0], sems.at[sem2]).wait()
        pltpu.make_async_copy(c2_hbm.at[i], buf[1], sems.at[sem2 + 1]).wait()

    start_fetch(0, bufs[0], 0)
    pltpu.sync_copy(z_hbm.at[pl.ds(base * _D, _SP * _D)], rv)

    def init_body(j, _):
        off = j * 16
        v = rv[pl.ds(off, 16)]
        rbv[pl.ds(off, 16)] = _rtne_bf16_value(v)
        qv[pl.ds(off, 16)] = jnp.zeros((16,), jnp.float32)
        return 0
    lax.fori_loop(0, _SP * _D // 16, init_body, 0)

    def process_codebook(i, ctv, c2v):
        pltpu.sync_copy(cbf_hbm.at[i], cbv)

        def group_body(g, ivecs):
            ivec0, ivec1 = ivecs
            pbase = g * _G
            rblanes = []
            for pp in range(_G):
                off = (pbase + pp) * _D
                rb1 = rbv[pl.ds(off, 16)]
                rb2 = rbv[pl.ds(off + 16, 16)]
                rblanes.append([rb1[j] for j in range(16)]
                               + [rb2[j] for j in range(16)])

            def kcb_body(kcb, carry):
                minv, mini = carry
                k0 = kcb * (16 * _KB)
                accs = []
                c2chunks = [c2v[pl.ds(k0 + kb * 16, 16)] for kb in range(_KB)]
                for pp in range(_G):
                    accs.append(list(c2chunks))
                for dd in range(_D):
                    ctvecs = [ctv[pl.ds(dd * _K + k0 + kb * 16, 16)]
                              for kb in range(_KB)]
                    for pp in range(_G):
                        s = rblanes[pp][dd]
                        for kb in range(_KB):
                            accs[pp][kb] = accs[pp][kb] + s * ctvecs[kb]
                new_minv, new_mini = [], []
                for pp in range(_G):
                    mv, mi = minv[pp], mini[pp]
                    for kb in range(_KB):
                        idxvec = (k0 + kb * 16) + lane
                        better = accs[pp][kb] < mv
                        mv = jnp.where(better, accs[pp][kb], mv)
                        mi = jnp.where(better, idxvec, mi)
                    new_minv.append(mv)
                    new_mini.append(mi)
                return (tuple(new_minv), tuple(new_mini))

            inf16 = jnp.full((16,), jnp.inf, jnp.float32)
            zero16 = jnp.zeros((16,), jnp.int32)
            minv, mini = lax.fori_loop(
                0, _NKCB, kcb_body,
                (tuple(inf16 for _ in range(_G)),
                 tuple(zero16 for _ in range(_G))))

            for pp in range(_G):
                mv, mi = minv[pp], mini[pp]
                for s in (8, 4, 2, 1):
                    rotf[pl.ds(0, 16)] = mv
                    rotf[pl.ds(16, 16)] = mv
                    roti[pl.ds(0, 16)] = mi
                    roti[pl.ds(16, 16)] = mi
                    rv_ = rotf[pl.ds(s, 16)]
                    ri_ = roti[pl.ds(s, 16)]
                    take = (rv_ < mv) | ((rv_ == mv) & (ri_ < mi))
                    mv = jnp.where(take, rv_, mv)
                    mi = jnp.where(take, ri_, mi)
                idxp = mi[0]
                p = pbase + pp
                ivec0 = jnp.where(lane == p, idxp, ivec0)
                ivec1 = jnp.where(lane == (p - 16), idxp, ivec1)
                off = p * _D
                coff = idxp * _D
                for h in range(2):
                    rvec = rv[pl.ds(off + h * 16, 16)]
                    cvec = cbv[pl.ds(coff + h * 16, 16)]
                    rnew = rvec - cvec
                    rv[pl.ds(off + h * 16, 16)] = rnew
                    rbv[pl.ds(off + h * 16, 16)] = _rtne_bf16_value(rnew)
                    qv[pl.ds(off + h * 16, 16)] = qv[pl.ds(off + h * 16, 16)] + cvec
            return (ivec0, ivec1)

        zero16 = jnp.zeros((16,), jnp.int32)
        ivec0, ivec1 = lax.fori_loop(0, _SP // _G, group_body,
                                     (zero16, zero16))
        idxv[pl.ds(i * _SP, 16)] = ivec0
        if _SP > 16:
            idxv[pl.ds(i * _SP + 16, 16)] = ivec1

    def pair_body(j, _):
        iA = 2 * j
        iB = iA + 1
        start_fetch(iB, bufs[1], 2)
        wait_fetch(iA, bufs[0], 0)
        process_codebook(iA, *bufs[0])

        @pl.when(j < _NCB // 2 - 1)
        def _prefetch_next():
            start_fetch(iA + 2, bufs[0], 0)

        wait_fetch(iB, bufs[1], 2)
        process_codebook(iB, *bufs[1])
        return 0

    lax.fori_loop(0, _NCB // 2, pair_body, 0)

    pltpu.sync_copy(qv, q_hbm.at[pl.ds(base * _D, _SP * _D)])
    for i in range(_NCB):
        pltpu.sync_copy(idxv.at[pl.ds(i * _SP, _SP)],
                        idx_hbm.at[pl.ds(i * _S + base, _SP)])


def kernel(z, codebooks):
    ct, c2, cbf = pl.pallas_call(
        _sc_prep_body,
        out_shape=[
            jax.ShapeDtypeStruct((_NCB, _D * _K), jnp.float32),
            jax.ShapeDtypeStruct((_NCB, _K), jnp.float32),
            jax.ShapeDtypeStruct((_NCB, _K * _D), jnp.float32),
        ],
    )(codebooks)

    sc_fn = pl.kernel(
        _sc_body,
        out_type=[
            jax.ShapeDtypeStruct((_S * _D,), jnp.float32),
            jax.ShapeDtypeStruct((_S * _NCB,), jnp.int32),
        ],
        mesh=plsc.VectorSubcoreMesh(core_axis_name="c", subcore_axis_name="s"),
        scratch_types=[
            pltpu.VMEM((_SP * _D,), jnp.float32),
            pltpu.VMEM((_SP * _D,), jnp.float32),
            pltpu.VMEM((_SP * _D,), jnp.float32),
            pltpu.VMEM((_D * _K,), jnp.float32),
            pltpu.VMEM((_K * _D,), jnp.float32),
            pltpu.VMEM((_K,), jnp.float32),
            pltpu.VMEM((_D * _K,), jnp.float32),
            pltpu.VMEM((_K,), jnp.float32),
            pltpu.VMEM((_SP * _NCB + 16,), jnp.int32),
            pltpu.VMEM((32,), jnp.float32),
            pltpu.VMEM((32,), jnp.int32),
            pltpu.SemaphoreType.DMA((4,)),
        ],
    )
    q_sc_flat, idx_sc_flat = sc_fn(
        z[_BTC:].reshape(-1), ct, c2, cbf)

    q_tc, idx_tc = pl.pallas_call(
        _rvq_body,
        grid=(_BTC // _BLK,),
        in_specs=[
            pl.BlockSpec((_BLK, _D), lambda i: (i, 0)),
            pl.BlockSpec((_NCB, _K, _D), lambda i: (0, 0, 0)),
        ],
        out_specs=[
            pl.BlockSpec((_BLK, _D), lambda i: (i, 0)),
            pl.BlockSpec((_BLK, _NCB), lambda i: (i, 0)),
        ],
        out_shape=[
            jax.ShapeDtypeStruct((_BTC, _D), jnp.float32),
            jax.ShapeDtypeStruct((_BTC, _NCB), jnp.int32),
        ],
    )(z[:_BTC], codebooks)

    q = jnp.concatenate([q_tc, q_sc_flat.reshape(_S, _D)], axis=0)
    idx_sc = idx_sc_flat.reshape(_NCB, _S).T
    idx = jnp.concatenate([idx_tc, idx_sc], axis=0)
    loss = jnp.zeros((), dtype=jnp.float32)
    return q, loss, idx.astype(jnp.int64)
